# baseline (speedup 1.0000x reference)
import functools

import jax
import jax.numpy as jnp
from jax import lax
from jax.experimental import pallas as pl
from jax.experimental.pallas import tpu as pltpu
from jax.experimental.pallas import tpu_sc as plsc

POOL_SIZE = 30
PROMPT_LEN = 5
TOP_K = 5
EMBED_DIM = 768
BATCH = 128
TOKENS = 196
BATCH_BLK = 16

_NC = 2
_NS = 16
_NW = _NC * _NS
_ROWS = BATCH * TOP_K
_ROW_W = PROMPT_LEN * EMBED_DIM
_RPW = _ROWS // _NW
_RPW_PAD = 24


def _topk_body(x_ref, prompt_ref, ids_ref):
    x = x_ref[...]
    embed_keys = jnp.max(x, axis=1) + 2.0 * (jnp.sum(x, axis=1) * (1.0 / TOKENS))

    keys = jnp.sum(prompt_ref[...], axis=1) * (1.0 / PROMPT_LEN)

    def l2(v):
        ss = jnp.sum(v * v, axis=1, keepdims=True)
        return v * lax.rsqrt(jnp.maximum(ss, 1e-12))

    sim = jnp.dot(l2(embed_keys), l2(keys).T,
                  preferred_element_type=jnp.float32)

    col = lax.broadcasted_iota(jnp.int32, sim.shape, 1)
    picked = []
    for _ in range(TOP_K):
        m = jnp.max(sim, axis=1, keepdims=True)
        idx = jnp.min(jnp.where(sim == m, col, POOL_SIZE), axis=1)
        picked.append(idx)
        sim = jnp.where(col == idx[:, None], -jnp.inf, sim)
    ids_ref[...] = jnp.stack(picked, axis=1)


def _topk_ids(input_embed, prompt):
    return pl.pallas_call(
        _topk_body,
        grid=(BATCH // BATCH_BLK,),
        in_specs=[
            pl.BlockSpec((BATCH_BLK, TOKENS, EMBED_DIM), lambda i: (i, 0, 0)),
            pl.BlockSpec((POOL_SIZE, PROMPT_LEN, EMBED_DIM), lambda i: (0, 0, 0)),
        ],
        out_specs=pl.BlockSpec((BATCH_BLK, TOP_K), lambda i: (i, 0)),
        out_shape=jax.ShapeDtypeStruct((BATCH, TOP_K), jnp.int32),
    )(input_embed, prompt)


def _sc_gather(table, idx2d):
    mesh = plsc.VectorSubcoreMesh(core_axis_name="c", subcore_axis_name="s")

    @functools.partial(
        pl.kernel,
        mesh=mesh,
        out_type=jax.ShapeDtypeStruct((_NW, _RPW_PAD, _ROW_W), jnp.float32),
        scratch_types=[
            pltpu.VMEM((_RPW_PAD,), jnp.int32),
            pltpu.VMEM((_RPW_PAD, _ROW_W), jnp.float32),
            pltpu.SemaphoreType.DMA,
        ],
    )
    def k(table_hbm, idx_hbm, out_hbm, idx_v, rows_v, sem):
        wid = lax.axis_index("s") * _NC + lax.axis_index("c")
        pltpu.sync_copy(idx_hbm.at[wid], idx_v)
        pltpu.async_copy(table_hbm.at[idx_v], rows_v, sem).wait()
        pltpu.sync_copy(rows_v, out_hbm.at[wid])

    return k(table, idx2d)


def kernel(input_embed, prompt):
    ids = _topk_ids(input_embed, prompt)
    idx2d = jnp.pad(ids.reshape(_NW, _RPW), ((0, 0), (0, _RPW_PAD - _RPW)))
    table = prompt.reshape(POOL_SIZE, _ROW_W)
    rows = _sc_gather(table, idx2d)
    return rows[:, :_RPW, :].reshape(BATCH, TOP_K * PROMPT_LEN, EMBED_DIM)
